# baseline (speedup 1.0000x reference)
import functools

import jax
import jax.numpy as jnp
from jax import lax
from jax.experimental import pallas as pl
from jax.experimental.pallas import tpu as pltpu

K_NN = 8
LANES = 128
GROUPS = 32

_SORT8 = [(0, 1), (2, 3), (4, 5), (6, 7),
          (0, 2), (1, 3), (4, 6), (5, 7),
          (1, 2), (5, 6),
          (0, 4), (1, 5), (2, 6), (3, 7),
          (2, 4), (3, 5),
          (1, 2), (3, 4), (5, 6)]

_BITONIC8 = [(0, 4), (1, 5), (2, 6), (3, 7),
             (0, 2), (1, 3), (4, 6), (5, 7),
             (0, 1), (2, 3), (4, 5), (6, 7)]


def _ce(v, p, i, j):
    c = v[i] <= v[j]
    lo = jnp.minimum(v[i], v[j])
    hi = jnp.maximum(v[i], v[j])
    plo = jnp.where(c, p[i], p[j])
    phi = jnp.where(c, p[j], p[i])
    v[i], v[j], p[i], p[j] = lo, hi, plo, phi


def _merge_low8(av, ap, bv, bp, resort):
    lv, lp = [], []
    for j in range(8):
        c = av[j] <= bv[7 - j]
        lv.append(jnp.minimum(av[j], bv[7 - j]))
        lp.append(jnp.where(c, ap[j], bp[7 - j]))
    if resort:
        for i, j in _BITONIC8:
            _ce(lv, lp, i, j)
    return lv, lp


def _knn_loss_kernel(x_rows_ref, x_all_ref, y_col_ref, y_row_ref, out_ref,
                     *, blk_r: int, n: int):
    i = pl.program_id(0)

    x_r = x_rows_ref[...]
    x_a = x_all_ref[...]

    sq_r = jnp.sum(x_r * x_r, axis=1, keepdims=True)
    sq_a = jnp.sum(x_a * x_a, axis=1, keepdims=True)
    ones_r = jnp.ones((x_r.shape[0], 1), jnp.float32)
    ones_a = jnp.ones((x_a.shape[0], 1), jnp.float32)
    aug_r = jnp.concatenate([x_r * -2.0, sq_r, ones_r], axis=1)
    aug_a = jnp.concatenate([x_a, ones_a, sq_a], axis=1)
    d2 = lax.dot_general(aug_r, aug_a, (((1,), (1,)), ((), ())),
                         preferred_element_type=jnp.float32)
    dist = jnp.sqrt(jnp.maximum(d2, 0.0))

    ci = lax.broadcasted_iota(jnp.int32, (blk_r, n), 1)
    row_g = lax.broadcasted_iota(jnp.int32, (blk_r, n), 0) + i * blk_r
    inf = jnp.float32(jnp.inf)
    dm = jnp.where(ci == row_g, inf, dist)

    denom = jnp.sum(jnp.exp(-dm), axis=1, keepdims=True)
    logden = jnp.log(denom)

    match = y_col_ref[...] == y_row_ref[...]
    pci = 2 * ci + jnp.where(match, 0, 1)
    big = jnp.int32(4 * n)

    v = [dm[:, g * LANES:(g + 1) * LANES] for g in range(GROUPS)]
    p = [pci[:, g * LANES:(g + 1) * LANES] for g in range(GROUPS)]
    quads = []
    for a in range(4):
        gv, gp = v[8 * a:8 * a + 8], p[8 * a:8 * a + 8]
        for ii, jj in _SORT8:
            _ce(gv, gp, ii, jj)
        quads.append((gv, gp))
    h0v, h0p = _merge_low8(quads[0][0], quads[0][1],
                           quads[1][0], quads[1][1], resort=True)
    h1v, h1p = _merge_low8(quads[2][0], quads[2][1],
                           quads[3][0], quads[3][1], resort=True)
    cv, cp = _merge_low8(h0v, h0p, h1v, h1p, resort=False)
    cand = jnp.concatenate(cv, axis=1)
    candp = jnp.concatenate(cp, axis=1)

    acc = jnp.zeros((blk_r, 1), jnp.float32)
    cnt = jnp.zeros((blk_r, 1), jnp.float32)
    for r in range(K_NN):
        m = jnp.min(cand, axis=1, keepdims=True)
        first = jnp.min(jnp.where(cand == m, candp, big), axis=1,
                        keepdims=True)
        mt = (1 - (first & 1)).astype(jnp.float32)
        acc = acc + mt * (m + logden)
        cnt = cnt + mt
        if r != K_NN - 1:
            cand = jnp.where(candp == first, inf, cand)

    per = jnp.where(cnt > 0.0, acc / jnp.maximum(cnt, 1.0), 0.0)
    partial = jnp.sum(per) / jnp.float32(n)

    out_ref[...] = jnp.full_like(out_ref, partial)


@jax.jit
def kernel(x, y):
    n, d = x.shape
    blk_r = 256
    y_col = y.reshape(n, 1)
    y_row = y.reshape(1, n)

    n_blk = n // blk_r
    out = pl.pallas_call(
        functools.partial(_knn_loss_kernel, blk_r=blk_r, n=n),
        grid=(n_blk,),
        in_specs=[
            pl.BlockSpec((blk_r, d), lambda i: (i, 0)),
            pl.BlockSpec((n, d), lambda i: (0, 0)),
            pl.BlockSpec((blk_r, 1), lambda i: (i, 0)),
            pl.BlockSpec((1, n), lambda i: (0, 0)),
        ],
        out_specs=pl.BlockSpec((1, 1, 1), lambda i: (i, 0, 0)),
        out_shape=jax.ShapeDtypeStruct((n_blk, 1, 1), jnp.float32),
        compiler_params=pltpu.CompilerParams(
            dimension_semantics=("parallel",)),
    )(x, x, y_col, y_row)
    return jnp.sum(out)


# speedup vs baseline: 1.0943x; 1.0943x over previous
import functools

import jax
import jax.numpy as jnp
from jax import lax
from jax.experimental import pallas as pl
from jax.experimental.pallas import tpu as pltpu

K_NN = 8
LANES = 128
GROUPS = 32

_SORT8 = [(0, 1), (2, 3), (4, 5), (6, 7),
          (0, 2), (1, 3), (4, 6), (5, 7),
          (1, 2), (5, 6),
          (0, 4), (1, 5), (2, 6), (3, 7),
          (2, 4), (3, 5),
          (1, 2), (3, 4), (5, 6)]

_BITONIC8 = [(0, 4), (1, 5), (2, 6), (3, 7),
             (0, 2), (1, 3), (4, 6), (5, 7),
             (0, 1), (2, 3), (4, 5), (6, 7)]


def _ce(v, p, i, j):
    c = v[i] <= v[j]
    lo = jnp.minimum(v[i], v[j])
    hi = jnp.maximum(v[i], v[j])
    plo = jnp.where(c, p[i], p[j])
    phi = jnp.where(c, p[j], p[i])
    v[i], v[j], p[i], p[j] = lo, hi, plo, phi


def _merge_low8(av, ap, bv, bp, resort):
    lv, lp = [], []
    for j in range(8):
        c = av[j] <= bv[7 - j]
        lv.append(jnp.minimum(av[j], bv[7 - j]))
        lp.append(jnp.where(c, ap[j], bp[7 - j]))
    if resort:
        for i, j in _BITONIC8:
            _ce(lv, lp, i, j)
    return lv, lp


def _knn_loss_kernel(x_rows_ref, x_all_ref, y_col_ref, y_row_ref, out_ref,
                     *, blk_r: int, n: int):
    i = pl.program_id(0)

    x_r = x_rows_ref[...]
    x_a = x_all_ref[...]

    dot = lax.dot_general(x_r * -2.0, x_a, (((1,), (1,)), ((), ())),
                          preferred_element_type=jnp.float32)
    sq_r = jnp.sum(x_r * x_r, axis=1, keepdims=True)
    ones_row = jnp.ones((1, x_a.shape[1]), dtype=jnp.float32)
    sq_a = lax.dot_general(ones_row, x_a * x_a, (((1,), (1,)), ((), ())),
                           preferred_element_type=jnp.float32)

    d2 = (sq_r + sq_a) + dot
    dist = jnp.sqrt(jnp.maximum(d2, 0.0))

    ci = lax.broadcasted_iota(jnp.int32, (blk_r, n), 1)
    row_g = lax.broadcasted_iota(jnp.int32, (blk_r, n), 0) + i * blk_r
    inf = jnp.float32(jnp.inf)
    dm = jnp.where(ci == row_g, inf, dist)

    denom = jnp.sum(jnp.exp(-dm), axis=1, keepdims=True)
    logden = jnp.log(denom)

    match = y_col_ref[...] == y_row_ref[...]
    pci = 2 * ci + jnp.where(match, 0, 1)
    big = jnp.int32(4 * n)

    v = [dm[:, g * LANES:(g + 1) * LANES] for g in range(GROUPS)]
    p = [pci[:, g * LANES:(g + 1) * LANES] for g in range(GROUPS)]
    quads = []
    for a in range(4):
        gv, gp = v[8 * a:8 * a + 8], p[8 * a:8 * a + 8]
        for ii, jj in _SORT8:
            _ce(gv, gp, ii, jj)
        quads.append((gv, gp))
    h0v, h0p = _merge_low8(quads[0][0], quads[0][1],
                           quads[1][0], quads[1][1], resort=True)
    h1v, h1p = _merge_low8(quads[2][0], quads[2][1],
                           quads[3][0], quads[3][1], resort=True)
    cv, cp = _merge_low8(h0v, h0p, h1v, h1p, resort=False)
    cand = jnp.concatenate(cv, axis=1)
    candp = jnp.concatenate(cp, axis=1)

    acc = jnp.zeros((blk_r, 1), jnp.float32)
    cnt = jnp.zeros((blk_r, 1), jnp.float32)
    for r in range(K_NN):
        m = jnp.min(cand, axis=1, keepdims=True)
        first = jnp.min(jnp.where(cand == m, candp, big), axis=1,
                        keepdims=True)
        mt = (1 - (first & 1)).astype(jnp.float32)
        acc = acc + mt * (m + logden)
        cnt = cnt + mt
        if r != K_NN - 1:
            cand = jnp.where(candp == first, inf, cand)

    per = jnp.where(cnt > 0.0, acc / jnp.maximum(cnt, 1.0), 0.0)
    partial = jnp.sum(per) / jnp.float32(n)

    out_ref[...] = jnp.full_like(out_ref, partial)


@jax.jit
def kernel(x, y):
    n, d = x.shape
    blk_r = 256
    y_col = y.reshape(n, 1)
    y_row = y.reshape(1, n)

    n_blk = n // blk_r
    out = pl.pallas_call(
        functools.partial(_knn_loss_kernel, blk_r=blk_r, n=n),
        grid=(n_blk,),
        in_specs=[
            pl.BlockSpec((blk_r, d), lambda i: (i, 0)),
            pl.BlockSpec((n, d), lambda i: (0, 0)),
            pl.BlockSpec((blk_r, 1), lambda i: (i, 0)),
            pl.BlockSpec((1, n), lambda i: (0, 0)),
        ],
        out_specs=pl.BlockSpec((1, 1, 1), lambda i: (i, 0, 0)),
        out_shape=jax.ShapeDtypeStruct((n_blk, 1, 1), jnp.float32),
        compiler_params=pltpu.CompilerParams(
            dimension_semantics=("parallel",)),
    )(x, x, y_col, y_row)
    return jnp.sum(out)


# speedup vs baseline: 1.1044x; 1.0092x over previous
import functools

import jax
import jax.numpy as jnp
from jax import lax
from jax.experimental import pallas as pl
from jax.experimental.pallas import tpu as pltpu

K_NN = 8
LANES = 128
GROUPS = 32

_SORT8 = [(0, 1), (2, 3), (4, 5), (6, 7),
          (0, 2), (1, 3), (4, 6), (5, 7),
          (1, 2), (5, 6),
          (0, 4), (1, 5), (2, 6), (3, 7),
          (2, 4), (3, 5),
          (1, 2), (3, 4), (5, 6)]

_BITONIC8 = [(0, 4), (1, 5), (2, 6), (3, 7),
             (0, 2), (1, 3), (4, 6), (5, 7),
             (0, 1), (2, 3), (4, 5), (6, 7)]


def _ce(v, p, i, j):
    c = v[i] <= v[j]
    lo = jnp.minimum(v[i], v[j])
    hi = jnp.maximum(v[i], v[j])
    plo = jnp.where(c, p[i], p[j])
    phi = jnp.where(c, p[j], p[i])
    v[i], v[j], p[i], p[j] = lo, hi, plo, phi


def _merge_low8(av, ap, bv, bp, resort):
    lv, lp = [], []
    for j in range(8):
        c = av[j] <= bv[7 - j]
        lv.append(jnp.minimum(av[j], bv[7 - j]))
        lp.append(jnp.where(c, ap[j], bp[7 - j]))
    if resort:
        for i, j in _BITONIC8:
            _ce(lv, lp, i, j)
    return lv, lp


def _knn_loss_kernel(x_rows_ref, x_all_ref, y_col_ref, y_row_ref, out_ref,
                     *, blk_r: int, n: int):
    i = pl.program_id(0)

    x_r = x_rows_ref[...]
    x_a = x_all_ref[...]

    dot = lax.dot_general(x_r * -2.0, x_a, (((1,), (1,)), ((), ())),
                          preferred_element_type=jnp.float32)
    sq_r = jnp.sum(x_r * x_r, axis=1, keepdims=True)
    ones_row = jnp.ones((1, x_a.shape[1]), dtype=jnp.float32)
    sq_a = lax.dot_general(ones_row, x_a * x_a, (((1,), (1,)), ((), ())),
                           preferred_element_type=jnp.float32)

    d2 = (sq_r + sq_a) + dot
    dist = jnp.sqrt(jnp.maximum(d2, 0.0))

    ci = lax.broadcasted_iota(jnp.int32, (blk_r, n), 1)
    row_g = lax.broadcasted_iota(jnp.int32, (blk_r, n), 0) + i * blk_r
    inf = jnp.float32(jnp.inf)
    dm = jnp.where(ci == row_g, inf, dist)

    denom = jnp.sum(jnp.exp(-dm), axis=1, keepdims=True)
    logden = jnp.log(denom)

    match = y_col_ref[...] == y_row_ref[...]
    pci = 2 * ci + jnp.where(match, 0, 1)
    big = jnp.int32(4 * n)

    v = [dm[:, g * LANES:(g + 1) * LANES] for g in range(GROUPS)]
    p = [pci[:, g * LANES:(g + 1) * LANES] for g in range(GROUPS)]
    quads = []
    for a in range(4):
        gv, gp = v[8 * a:8 * a + 8], p[8 * a:8 * a + 8]
        for ii, jj in _SORT8:
            _ce(gv, gp, ii, jj)
        quads.append((gv, gp))
    h0v, h0p = _merge_low8(quads[0][0], quads[0][1],
                           quads[1][0], quads[1][1], resort=True)
    h1v, h1p = _merge_low8(quads[2][0], quads[2][1],
                           quads[3][0], quads[3][1], resort=True)
    cv, cp = _merge_low8(h0v, h0p, h1v, h1p, resort=False)
    cand = jnp.concatenate(cv, axis=1)
    candp = jnp.concatenate(cp, axis=1)

    acc = jnp.zeros((blk_r, 1), jnp.float32)
    cnt = jnp.zeros((blk_r, 1), jnp.float32)
    for r in range(K_NN):
        m = jnp.min(cand, axis=1, keepdims=True)
        first = jnp.min(jnp.where(cand == m, candp, big), axis=1,
                        keepdims=True)
        mt = (1 - (first & 1)).astype(jnp.float32)
        acc = acc + mt * (m + logden)
        cnt = cnt + mt
        if r != K_NN - 1:
            cand = jnp.where(candp == first, inf, cand)

    per = jnp.where(cnt > 0.0, acc / jnp.maximum(cnt, 1.0), 0.0)
    partial = jnp.sum(per) / jnp.float32(n)

    @pl.when(i == 0)
    def _():
        out_ref[...] = jnp.zeros_like(out_ref)

    out_ref[...] = out_ref[...] + partial


@jax.jit
def kernel(x, y):
    n, d = x.shape
    blk_r = 256
    y_col = y.reshape(n, 1)
    y_row = y.reshape(1, n)

    n_blk = n // blk_r
    out = pl.pallas_call(
        functools.partial(_knn_loss_kernel, blk_r=blk_r, n=n),
        grid=(n_blk,),
        in_specs=[
            pl.BlockSpec((blk_r, d), lambda i: (i, 0)),
            pl.BlockSpec((n, d), lambda i: (0, 0)),
            pl.BlockSpec((blk_r, 1), lambda i: (i, 0)),
            pl.BlockSpec((1, n), lambda i: (0, 0)),
        ],
        out_specs=pl.BlockSpec((1, 1), lambda i: (0, 0)),
        out_shape=jax.ShapeDtypeStruct((1, 1), jnp.float32),
    )(x, x, y_col, y_row)
    return out[0, 0]


# speedup vs baseline: 1.1484x; 1.0398x over previous
import functools

import jax
import jax.numpy as jnp
from jax import lax
from jax.experimental import pallas as pl
from jax.experimental.pallas import tpu as pltpu

K_NN = 8
LANES = 128
GROUPS = 32

_SORT8 = [(0, 1), (2, 3), (4, 5), (6, 7),
          (0, 2), (1, 3), (4, 6), (5, 7),
          (1, 2), (5, 6),
          (0, 4), (1, 5), (2, 6), (3, 7),
          (2, 4), (3, 5),
          (1, 2), (3, 4), (5, 6)]

_BITONIC8 = [(0, 4), (1, 5), (2, 6), (3, 7),
             (0, 2), (1, 3), (4, 6), (5, 7),
             (0, 1), (2, 3), (4, 5), (6, 7)]


def _ce(v, p, i, j):
    c = v[i] <= v[j]
    lo = jnp.minimum(v[i], v[j])
    hi = jnp.maximum(v[i], v[j])
    plo = jnp.where(c, p[i], p[j])
    phi = jnp.where(c, p[j], p[i])
    v[i], v[j], p[i], p[j] = lo, hi, plo, phi


def _merge_low8(av, ap, bv, bp, resort):
    lv, lp = [], []
    for j in range(8):
        c = av[j] <= bv[7 - j]
        lv.append(jnp.minimum(av[j], bv[7 - j]))
        lp.append(jnp.where(c, ap[j], bp[7 - j]))
    if resort:
        for i, j in _BITONIC8:
            _ce(lv, lp, i, j)
    return lv, lp


def _knn_loss_kernel(x_rows_ref, x_all_ref, y_col_ref, y_row_ref, out_ref,
                     *, blk_r: int, n: int):
    i = pl.program_id(0)

    x_r = x_rows_ref[...]
    x_a = x_all_ref[...]

    dot = lax.dot_general(x_r * -2.0, x_a, (((1,), (1,)), ((), ())),
                          preferred_element_type=jnp.float32)
    sq_r = jnp.sum(x_r * x_r, axis=1, keepdims=True)
    ones_row = jnp.ones((1, x_a.shape[1]), dtype=jnp.float32)
    sq_a = lax.dot_general(ones_row, x_a * x_a, (((1,), (1,)), ((), ())),
                           preferred_element_type=jnp.float32)

    d2 = (sq_r + sq_a) + dot
    dist = jnp.sqrt(jnp.maximum(d2, 0.0))

    ci = lax.broadcasted_iota(jnp.int32, (blk_r, n), 1)
    row_g = lax.broadcasted_iota(jnp.int32, (blk_r, n), 0) + i * blk_r
    inf = jnp.float32(jnp.inf)
    dm = jnp.where(ci == row_g, inf, dist)

    denom = jnp.sum(jnp.exp(-dm), axis=1, keepdims=True)
    logden = jnp.log(denom)

    match = y_col_ref[...] == y_row_ref[...]
    pci = 2 * ci + jnp.where(match, 0, 1)
    big = jnp.int32(4 * n)

    v = [dm[:, g * LANES:(g + 1) * LANES] for g in range(GROUPS)]
    p = [pci[:, g * LANES:(g + 1) * LANES] for g in range(GROUPS)]
    quads = []
    for a in range(4):
        gv, gp = v[8 * a:8 * a + 8], p[8 * a:8 * a + 8]
        for ii, jj in _SORT8:
            _ce(gv, gp, ii, jj)
        quads.append((gv, gp))
    h0v, h0p = _merge_low8(quads[0][0], quads[0][1],
                           quads[1][0], quads[1][1], resort=True)
    h1v, h1p = _merge_low8(quads[2][0], quads[2][1],
                           quads[3][0], quads[3][1], resort=True)
    cv, cp = _merge_low8(h0v, h0p, h1v, h1p, resort=False)
    cand = jnp.concatenate(cv, axis=1)
    candp = jnp.concatenate(cp, axis=1)

    acc = jnp.zeros((blk_r, 1), jnp.float32)
    cnt = jnp.zeros((blk_r, 1), jnp.float32)
    for r in range(K_NN):
        m = jnp.min(cand, axis=1, keepdims=True)
        first = jnp.min(jnp.where(cand == m, candp, big), axis=1,
                        keepdims=True)
        mt = (1 - (first & 1)).astype(jnp.float32)
        acc = acc + mt * (m + logden)
        cnt = cnt + mt
        if r != K_NN - 1:
            cand = jnp.where(candp == first, inf, cand)

    per = jnp.where(cnt > 0.0, acc / jnp.maximum(cnt, 1.0), 0.0)
    partial = jnp.sum(per) / jnp.float32(n)

    @pl.when(i == 0)
    def _():
        out_ref[...] = jnp.zeros_like(out_ref)

    out_ref[...] = out_ref[...] + partial


@jax.jit
def kernel(x, y):
    n, d = x.shape
    blk_r = 512
    y_col = y.reshape(n, 1)
    y_row = y.reshape(1, n)

    n_blk = n // blk_r
    out = pl.pallas_call(
        functools.partial(_knn_loss_kernel, blk_r=blk_r, n=n),
        grid=(n_blk,),
        in_specs=[
            pl.BlockSpec((blk_r, d), lambda i: (i, 0)),
            pl.BlockSpec((n, d), lambda i: (0, 0)),
            pl.BlockSpec((blk_r, 1), lambda i: (i, 0)),
            pl.BlockSpec((1, n), lambda i: (0, 0)),
        ],
        out_specs=pl.BlockSpec((1, 1), lambda i: (0, 0)),
        out_shape=jax.ShapeDtypeStruct((1, 1), jnp.float32),
    )(x, x, y_col, y_row)
    return out[0, 0]


# speedup vs baseline: 1.1671x; 1.0163x over previous
import functools

import jax
import jax.numpy as jnp
from jax import lax
from jax.experimental import pallas as pl
from jax.experimental.pallas import tpu as pltpu

K_NN = 8
LANES = 128
GROUPS = 32

_SORT8 = [(0, 1), (2, 3), (4, 5), (6, 7),
          (0, 2), (1, 3), (4, 6), (5, 7),
          (1, 2), (5, 6),
          (0, 4), (1, 5), (2, 6), (3, 7),
          (2, 4), (3, 5),
          (1, 2), (3, 4), (5, 6)]

_BITONIC8 = [(0, 4), (1, 5), (2, 6), (3, 7),
             (0, 2), (1, 3), (4, 6), (5, 7),
             (0, 1), (2, 3), (4, 5), (6, 7)]


def _ce(v, p, i, j):
    c = v[i] <= v[j]
    lo = jnp.minimum(v[i], v[j])
    hi = jnp.maximum(v[i], v[j])
    plo = jnp.where(c, p[i], p[j])
    phi = jnp.where(c, p[j], p[i])
    v[i], v[j], p[i], p[j] = lo, hi, plo, phi


def _merge_low8(av, ap, bv, bp, resort):
    lv, lp = [], []
    for j in range(8):
        c = av[j] <= bv[7 - j]
        lv.append(jnp.minimum(av[j], bv[7 - j]))
        lp.append(jnp.where(c, ap[j], bp[7 - j]))
    if resort:
        for i, j in _BITONIC8:
            _ce(lv, lp, i, j)
    return lv, lp


def _knn_loss_kernel(x_rows_ref, x_all_ref, y_col_ref, y_row_ref, out_ref,
                     *, blk_r: int, n: int):
    i = pl.program_id(0)

    x_r = x_rows_ref[...]
    x_a = x_all_ref[...]

    dot = lax.dot_general(x_r * -2.0, x_a, (((1,), (1,)), ((), ())),
                          preferred_element_type=jnp.float32)
    sq_r = jnp.sum(x_r * x_r, axis=1, keepdims=True)
    ones_row = jnp.ones((1, x_a.shape[1]), dtype=jnp.float32)
    sq_a = lax.dot_general(ones_row, x_a * x_a, (((1,), (1,)), ((), ())),
                           preferred_element_type=jnp.float32)

    d2 = (sq_r + sq_a) + dot
    dist = jnp.sqrt(jnp.maximum(d2, 0.0))

    ci = lax.broadcasted_iota(jnp.int32, (blk_r, n), 1)
    row_g = lax.broadcasted_iota(jnp.int32, (blk_r, n), 0) + i * blk_r
    inf = jnp.float32(jnp.inf)
    dm = jnp.where(ci == row_g, inf, dist)

    denom = jnp.sum(jnp.exp(-dm), axis=1, keepdims=True)
    logden = jnp.log(denom)

    match = y_col_ref[...] == y_row_ref[...]
    pci = 2 * ci + jnp.where(match, 0, 1)
    big = jnp.int32(4 * n)

    v = [dm[:, g * LANES:(g + 1) * LANES] for g in range(GROUPS)]
    p = [pci[:, g * LANES:(g + 1) * LANES] for g in range(GROUPS)]
    quads = []
    for a in range(4):
        gv, gp = v[8 * a:8 * a + 8], p[8 * a:8 * a + 8]
        for ii, jj in _SORT8:
            _ce(gv, gp, ii, jj)
        quads.append((gv, gp))
    h0v, h0p = _merge_low8(quads[0][0], quads[0][1],
                           quads[1][0], quads[1][1], resort=True)
    h1v, h1p = _merge_low8(quads[2][0], quads[2][1],
                           quads[3][0], quads[3][1], resort=True)
    cv, cp = _merge_low8(h0v, h0p, h1v, h1p, resort=False)
    cand = jnp.concatenate(cv, axis=1)
    candp = jnp.concatenate(cp, axis=1)

    acc = jnp.zeros((blk_r, 1), jnp.float32)
    cnt = jnp.zeros((blk_r, 1), jnp.float32)
    for r in range(K_NN):
        m = jnp.min(cand, axis=1, keepdims=True)
        first = jnp.min(jnp.where(cand == m, candp, big), axis=1,
                        keepdims=True)
        mt = (1 - (first & 1)).astype(jnp.float32)
        acc = acc + mt * (m + logden)
        cnt = cnt + mt
        if r != K_NN - 1:
            cand = jnp.where(candp == first, inf, cand)

    per = jnp.where(cnt > 0.0, acc / jnp.maximum(cnt, 1.0), 0.0)
    partial = jnp.sum(per) / jnp.float32(n)

    @pl.when(i == 0)
    def _():
        out_ref[...] = jnp.zeros_like(out_ref)

    out_ref[...] = out_ref[...] + partial


@jax.jit
def kernel(x, y):
    n, d = x.shape
    blk_r = 1024
    y_col = y.reshape(n, 1)
    y_row = y.reshape(1, n)

    n_blk = n // blk_r
    out = pl.pallas_call(
        functools.partial(_knn_loss_kernel, blk_r=blk_r, n=n),
        grid=(n_blk,),
        in_specs=[
            pl.BlockSpec((blk_r, d), lambda i: (i, 0)),
            pl.BlockSpec((n, d), lambda i: (0, 0)),
            pl.BlockSpec((blk_r, 1), lambda i: (i, 0)),
            pl.BlockSpec((1, n), lambda i: (0, 0)),
        ],
        out_specs=pl.BlockSpec((1, 1), lambda i: (0, 0)),
        out_shape=jax.ShapeDtypeStruct((1, 1), jnp.float32),
    )(x, x, y_col, y_row)
    return out[0, 0]
